# baseline (speedup 1.0000x reference)
import functools

import jax
import jax.numpy as jnp
from jax import lax
from jax.experimental import pallas as pl
from jax.experimental.pallas import tpu as pltpu
from jax.experimental.pallas import tpu_sc as plsc

B = 32
SEQ = 512
D = 512
KB_N = 100000
TOP_N = 5
TOP_K = 5

_NEG = -3.0e38
_BIGI = 2 ** 30


def _sc_gather(table, idx):
    n = idx.shape[0]
    nw = 32
    per_w = n // nw
    ch = 128
    mesh = plsc.VectorSubcoreMesh(core_axis_name="c", subcore_axis_name="s")

    @functools.partial(
        pl.kernel,
        mesh=mesh,
        out_type=jax.ShapeDtypeStruct((n, D), jnp.float32),
        scratch_types=[
            pltpu.VMEM((ch,), jnp.int32),
            pltpu.VMEM((ch, D), jnp.float32),
            pltpu.SemaphoreType.DMA,
        ],
    )
    def k(table_hbm, idx_hbm, out_hbm, idx_v, rows_v, sem):
        wid = lax.axis_index("s") * 2 + lax.axis_index("c")
        base = wid * per_w
        for j in range(per_w // ch):
            off = base + j * ch
            pltpu.sync_copy(idx_hbm.at[pl.ds(off, ch)], idx_v)
            pltpu.async_copy(table_hbm.at[idx_v], rows_v, sem).wait()
            pltpu.sync_copy(rows_v, out_hbm.at[pl.ds(off, ch)])

    return k(table, idx)


def _lane_sum_512(x):
    c = ((x[:, 0:128] + x[:, 128:256]) + x[:, 256:384]) + x[:, 384:512]
    acc = c[:, 0:8]
    for k in range(1, 16):
        acc = acc + c[:, 8 * k:8 * (k + 1)]
    h = acc[:, 0:4] + acc[:, 4:8]
    h = h[:, 0:2] + h[:, 2:4]
    return h[:, 0:1] + h[:, 1:2]


def _lane_sum_512_T(x):
    r = x.shape[0]
    c = ((x[:, 0:128] + x[:, 128:256]) + x[:, 256:384]) + x[:, 384:512]
    ct = jnp.transpose(c)
    acc = ct[0:8, :]
    for k in range(1, 16):
        acc = acc + ct[8 * k:8 * (k + 1), :]
    h = acc[0:4, :] + acc[4:8, :]
    h = h[0:2, :] + h[2:4, :]
    tot = h[0:1, :] + h[1:2, :]
    back = jnp.transpose(jnp.broadcast_to(tot, (8, r)))
    return back[:, 0:1]


def _col_to_row(x):
    return jnp.transpose(jnp.broadcast_to(x, (x.shape[0], 8)))[0:1, :]


def _row_to_col(x):
    return jnp.transpose(jnp.broadcast_to(x, (8, x.shape[1])))[:, 0:1]


def _stage2_compute(emb_ref, tok_ref, w_ref, b_ref, recon_ref, score_ref, qn_ref):
    e = emb_ref[0]
    g = lax.dot_general(e, e, (((1,), (1,)), ((), ())),
                        preferred_element_type=jnp.float32)
    rowi = lax.broadcasted_iota(jnp.int32, (SEQ, SEQ), 0)
    coli = lax.broadcasted_iota(jnp.int32, (SEQ, SEQ), 1)
    eye = rowi == coli
    sq_col = _lane_sum_512_T(e * e)
    sq_row = _col_to_row(sq_col)
    d2 = sq_col + sq_row - 2.0 * g

    dsel = []
    work = d2
    for _ in range(TOP_N):
        m = jnp.min(work, axis=1, keepdims=True)
        dsel.append(jnp.sqrt(jnp.maximum(m, 0.0)))
        first = jnp.min(jnp.where(work <= m, coli, _BIGI), axis=1, keepdims=True)
        work = jnp.where(coli == first, 3.0e38, work)
    mean_dist = (((dsel[0] + dsel[4]) + dsel[2]) + (dsel[1] + dsel[3])
                 ) * jnp.float32(0.2)

    lin = lax.dot_general(e, w_ref[...], (((1,), (0,)), ((), ())),
                          preferred_element_type=jnp.float32) + b_ref[0, 0]
    s_col = jax.nn.sigmoid(lin) * jnp.exp(-mean_dist)

    s_row = _col_to_row(s_col)
    tok_row = tok_ref[0]
    tok_col = _row_to_col(tok_row)

    beats = (s_row > s_col) | ((s_row == s_col) & (coli < rowi))
    rank_col = jnp.sum(beats.astype(jnp.int32), axis=1, keepdims=True)

    recon_row = jnp.sum(jnp.where(rank_col == coli, tok_col, 0),
                        axis=0, keepdims=True)

    acc8 = e[0:8, :]
    for k in range(1, 64):
        acc8 = acc8 + e[8 * k:8 * (k + 1), :]
    acc4 = acc8[0:4, :] + acc8[4:8, :]
    acc2 = acc4[0:2, :] + acc4[2:4, :]
    qm = (acc2[0:1, :] + acc2[1:2, :]) * jnp.float32(1.0 / 512.0)
    nrm = jnp.sqrt(_lane_sum_512(qm * qm))
    qn = qm / (nrm + 1e-8)

    recon_ref[0] = recon_row
    score_ref[0] = s_row
    qn_ref[pl.ds(pl.program_id(0), 1), :] = qn


_CHUNK = 5000
_NCHUNK = KB_N // _CHUNK


def _stage3_compute(c, qn, kb_ref, idx_out_ref, bv_ref, bi_ref):
    @pl.when(c == 0)
    def _init():
        bv_ref[...] = jnp.full((B, 16), _NEG, jnp.float32)
        bi_ref[...] = jnp.zeros((B, 16), jnp.int32)

    kbc = kb_ref[...]
    n2 = _lane_sum_512_T(kbc * kbc)
    kbn = kbc / (jnp.sqrt(n2) + 1e-8)
    s = lax.dot_general(qn, kbn, (((1,), (1,)), ((), ())),
                        preferred_element_type=jnp.float32)

    coli = lax.broadcasted_iota(jnp.int32, (B, _CHUNK), 1)
    lane16 = lax.broadcasted_iota(jnp.int32, (B, 16), 1)

    cand_v = bv_ref[...]
    cand_i = bi_ref[...]
    base = c * _CHUNK
    for t in range(TOP_K):
        m = jnp.max(s, axis=1, keepdims=True)
        first = jnp.min(jnp.where(s >= m, coli, _BIGI), axis=1, keepdims=True)
        s = jnp.where(coli == first, _NEG, s)
        cand_v = jnp.where(lane16 == 8 + t, m, cand_v)
        cand_i = jnp.where(lane16 == 8 + t, base + first, cand_i)

    res_v = jnp.full((B, 16), _NEG, jnp.float32)
    res_i = jnp.zeros((B, 16), jnp.int32)
    work = cand_v
    for t in range(TOP_K):
        m = jnp.max(work, axis=1, keepdims=True)
        sel = jnp.min(jnp.where(work >= m, cand_i, _BIGI), axis=1, keepdims=True)
        res_v = jnp.where(lane16 == t, m, res_v)
        res_i = jnp.where(lane16 == t, sel, res_i)
        work = jnp.where((work >= m) & (cand_i == sel), _NEG, work)
    bv_ref[...] = res_v
    bi_ref[...] = res_i
    idx_out_ref[...] = res_i


def _fused_body(emb_ref, tok_ref, w_ref, b_ref, kb_ref,
                recon_ref, score_ref, idx_out_ref,
                qn_s, bv_ref, bi_ref):
    i = pl.program_id(0)

    @pl.when(i < B)
    def _phase_a():
        _stage2_compute(emb_ref, tok_ref, w_ref, b_ref,
                        recon_ref, score_ref, qn_s)

    @pl.when(i >= B)
    def _phase_b():
        _stage3_compute(i - B, qn_s[...], kb_ref, idx_out_ref, bv_ref, bi_ref)


def _fused(emb, tok3, w, b11, kb):
    nb = B - 1

    return pl.pallas_call(
        _fused_body,
        grid=(B + _NCHUNK,),
        in_specs=[
            pl.BlockSpec((1, SEQ, D), lambda i: (jnp.minimum(i, nb), 0, 0)),
            pl.BlockSpec((1, 1, SEQ), lambda i: (jnp.minimum(i, nb), 0, 0)),
            pl.BlockSpec((D, 1), lambda i: (0, 0)),
            pl.BlockSpec((1, 1), lambda i: (0, 0)),
            pl.BlockSpec((_CHUNK, D),
                         lambda i: (jnp.clip(i - B, 0, _NCHUNK - 1), 0)),
        ],
        out_specs=[
            pl.BlockSpec((1, 1, SEQ), lambda i: (jnp.minimum(i, nb), 0, 0)),
            pl.BlockSpec((1, 1, SEQ), lambda i: (jnp.minimum(i, nb), 0, 0)),
            pl.BlockSpec((B, 16), lambda i: (0, 0)),
        ],
        out_shape=[
            jax.ShapeDtypeStruct((B, 1, SEQ), jnp.int32),
            jax.ShapeDtypeStruct((B, 1, SEQ), jnp.float32),
            jax.ShapeDtypeStruct((B, 16), jnp.int32),
        ],
        scratch_shapes=[
            pltpu.VMEM((B, D), jnp.float32),
            pltpu.VMEM((B, 16), jnp.float32),
            pltpu.VMEM((B, 16), jnp.int32),
        ],
    )(emb, tok3, w, b11, kb)


def kernel(query_tokens, prompt_tokens, rag_tokens, embedding, W_score, b_score, kb):
    tok_dtype = query_tokens.dtype
    combined = jnp.concatenate([query_tokens[:, :256], rag_tokens], axis=1)
    comb32 = combined.astype(jnp.int32)

    emb = _sc_gather(embedding, comb32.reshape(-1)).reshape(B, SEQ, D)

    recon, scores, idx = _fused(
        emb,
        comb32.reshape(B, 1, SEQ),
        W_score,
        b_score.reshape(1, 1),
        kb,
    )

    return (
        recon.reshape(B, SEQ).astype(tok_dtype),
        scores.reshape(B, SEQ),
        idx[:, :TOP_K],
    )


# speedup vs baseline: 1.0082x; 1.0082x over previous
import functools

import jax
import jax.numpy as jnp
from jax import lax
from jax.experimental import pallas as pl
from jax.experimental.pallas import tpu as pltpu
from jax.experimental.pallas import tpu_sc as plsc

B = 32
SEQ = 512
D = 512
KB_N = 100000
TOP_N = 5
TOP_K = 5

_NEG = -3.0e38
_BIGI = 2 ** 30


def _sc_gather(table, idx):
    n = idx.shape[0]
    nw = 32
    per_w = n // nw
    ch = 128
    mesh = plsc.VectorSubcoreMesh(core_axis_name="c", subcore_axis_name="s")

    @functools.partial(
        pl.kernel,
        mesh=mesh,
        out_type=jax.ShapeDtypeStruct((n, D), jnp.float32),
        scratch_types=[
            pltpu.VMEM((ch,), jnp.int32),
            pltpu.VMEM((ch, D), jnp.float32),
            pltpu.SemaphoreType.DMA,
        ],
    )
    def k(table_hbm, idx_hbm, out_hbm, idx_v, rows_v, sem):
        wid = lax.axis_index("s") * 2 + lax.axis_index("c")
        base = wid * per_w
        for j in range(per_w // ch):
            off = base + j * ch
            pltpu.sync_copy(idx_hbm.at[pl.ds(off, ch)], idx_v)
            pltpu.async_copy(table_hbm.at[idx_v], rows_v, sem).wait()
            pltpu.sync_copy(rows_v, out_hbm.at[pl.ds(off, ch)])

    return k(table, idx)


def _lane_sum_512(x):
    c = ((x[:, 0:128] + x[:, 128:256]) + x[:, 256:384]) + x[:, 384:512]
    acc = c[:, 0:8]
    for k in range(1, 16):
        acc = acc + c[:, 8 * k:8 * (k + 1)]
    h = acc[:, 0:4] + acc[:, 4:8]
    h = h[:, 0:2] + h[:, 2:4]
    return h[:, 0:1] + h[:, 1:2]


def _lane_sum_512_T(x):
    r = x.shape[0]
    c = ((x[:, 0:128] + x[:, 128:256]) + x[:, 256:384]) + x[:, 384:512]
    ct = jnp.transpose(c)
    acc = ct[0:8, :]
    for k in range(1, 16):
        acc = acc + ct[8 * k:8 * (k + 1), :]
    h = acc[0:4, :] + acc[4:8, :]
    h = h[0:2, :] + h[2:4, :]
    tot = h[0:1, :] + h[1:2, :]
    back = jnp.transpose(jnp.broadcast_to(tot, (8, r)))
    return back[:, 0:1]


def _col_to_row(x):
    return jnp.transpose(jnp.broadcast_to(x, (x.shape[0], 8)))[0:1, :]


def _row_to_col(x):
    return jnp.transpose(jnp.broadcast_to(x, (8, x.shape[1])))[:, 0:1]


def _stage2_compute(emb_ref, tok_ref, w_ref, b_ref, recon_ref, score_ref, qn_ref):
    e = emb_ref[0]
    g = lax.dot_general(e, e, (((1,), (1,)), ((), ())),
                        preferred_element_type=jnp.float32)
    rowi = lax.broadcasted_iota(jnp.int32, (SEQ, SEQ), 0)
    coli = lax.broadcasted_iota(jnp.int32, (SEQ, SEQ), 1)
    eye = rowi == coli
    sq_col = _lane_sum_512_T(e * e)
    sq_row = _col_to_row(sq_col)
    d2 = sq_col + sq_row - 2.0 * g

    dsel = []
    work = d2
    for _ in range(TOP_N):
        m = jnp.min(work, axis=1, keepdims=True)
        dsel.append(jnp.sqrt(jnp.maximum(m, 0.0)))
        first = jnp.min(jnp.where(work <= m, coli, _BIGI), axis=1, keepdims=True)
        work = jnp.where(coli == first, 3.0e38, work)
    mean_dist = (((dsel[0] + dsel[4]) + dsel[2]) + (dsel[1] + dsel[3])
                 ) * jnp.float32(0.2)

    lin = lax.dot_general(e, w_ref[...], (((1,), (0,)), ((), ())),
                          preferred_element_type=jnp.float32) + b_ref[0, 0]
    s_col = jax.nn.sigmoid(lin) * jnp.exp(-mean_dist)

    s_row = _col_to_row(s_col)
    tok_row = tok_ref[0]
    tok_col = _row_to_col(tok_row)

    beats = (s_row > s_col) | ((s_row == s_col) & (coli < rowi))
    rank_col = jnp.sum(beats.astype(jnp.int32), axis=1, keepdims=True)

    recon_row = jnp.sum(jnp.where(rank_col == coli, tok_col, 0),
                        axis=0, keepdims=True)

    acc8 = e[0:8, :]
    for k in range(1, 64):
        acc8 = acc8 + e[8 * k:8 * (k + 1), :]
    acc4 = acc8[0:4, :] + acc8[4:8, :]
    acc2 = acc4[0:2, :] + acc4[2:4, :]
    qm = (acc2[0:1, :] + acc2[1:2, :]) * jnp.float32(1.0 / 512.0)
    nrm = jnp.sqrt(_lane_sum_512(qm * qm))
    qn = qm / (nrm + 1e-8)

    recon_ref[0] = recon_row
    score_ref[0] = s_row
    qn_ref[pl.ds(pl.program_id(0), 1), :] = qn


_CHUNK = 5000
_NCHUNK = KB_N // _CHUNK


def _stage3_compute(c, qn, kb_ref, idx_out_ref, bv_ref, bi_ref):
    @pl.when(c == 0)
    def _init():
        bv_ref[...] = jnp.full((B, 16), _NEG, jnp.float32)
        bi_ref[...] = jnp.zeros((B, 16), jnp.int32)

    kbc = kb_ref[...]
    n2 = _lane_sum_512_T(kbc * kbc)
    kbn = kbc / (jnp.sqrt(n2) + 1e-8)
    s = lax.dot_general(qn, kbn, (((1,), (1,)), ((), ())),
                        preferred_element_type=jnp.float32)

    coli = lax.broadcasted_iota(jnp.int32, (B, _CHUNK), 1)
    lane16 = lax.broadcasted_iota(jnp.int32, (B, 16), 1)

    cand_v = bv_ref[...]
    cand_i = bi_ref[...]
    base = c * _CHUNK
    for t in range(TOP_K):
        m = jnp.max(s, axis=1, keepdims=True)
        first = jnp.min(jnp.where(s >= m, coli, _BIGI), axis=1, keepdims=True)
        s = jnp.where(coli == first, _NEG, s)
        cand_v = jnp.where(lane16 == 8 + t, m, cand_v)
        cand_i = jnp.where(lane16 == 8 + t, base + first, cand_i)

    res_v = jnp.full((B, 16), _NEG, jnp.float32)
    res_i = jnp.zeros((B, 16), jnp.int32)
    work = cand_v
    for t in range(TOP_K):
        m = jnp.max(work, axis=1, keepdims=True)
        sel = jnp.min(jnp.where(work >= m, cand_i, _BIGI), axis=1, keepdims=True)
        res_v = jnp.where(lane16 == t, m, res_v)
        res_i = jnp.where(lane16 == t, sel, res_i)
        work = jnp.where((work >= m) & (cand_i == sel), _NEG, work)
    bv_ref[...] = res_v
    bi_ref[...] = res_i
    idx_out_ref[...] = res_i


def _stage2_body(emb_ref, tok_ref, w_ref, b_ref, recon_ref, score_ref, qn_ref):
    _stage2_compute(emb_ref, tok_ref, w_ref, b_ref, recon_ref, score_ref, qn_ref)


def _stage2(emb, tok3, w, b11):
    return pl.pallas_call(
        _stage2_body,
        grid=(B,),
        in_specs=[
            pl.BlockSpec((1, SEQ, D), lambda b: (b, 0, 0)),
            pl.BlockSpec((1, 1, SEQ), lambda b: (b, 0, 0)),
            pl.BlockSpec((D, 1), lambda b: (0, 0)),
            pl.BlockSpec((1, 1), lambda b: (0, 0)),
        ],
        out_specs=[
            pl.BlockSpec((1, 1, SEQ), lambda b: (b, 0, 0)),
            pl.BlockSpec((1, 1, SEQ), lambda b: (b, 0, 0)),
            pl.BlockSpec((B, D), lambda b: (0, 0)),
        ],
        out_shape=[
            jax.ShapeDtypeStruct((B, 1, SEQ), jnp.int32),
            jax.ShapeDtypeStruct((B, 1, SEQ), jnp.float32),
            jax.ShapeDtypeStruct((B, D), jnp.float32),
        ],
    )(emb, tok3, w, b11)


def _stage3_body(qn_ref, kb_ref, idx_out_ref, bv_ref, bi_ref):
    _stage3_compute(pl.program_id(0), qn_ref[...], kb_ref,
                    idx_out_ref, bv_ref, bi_ref)


def _stage3(qn, kb):
    return pl.pallas_call(
        _stage3_body,
        grid=(_NCHUNK,),
        in_specs=[
            pl.BlockSpec((B, D), lambda c: (0, 0)),
            pl.BlockSpec((_CHUNK, D), lambda c: (c, 0)),
        ],
        out_specs=pl.BlockSpec((B, 16), lambda c: (0, 0)),
        out_shape=jax.ShapeDtypeStruct((B, 16), jnp.int32),
        scratch_shapes=[
            pltpu.VMEM((B, 16), jnp.float32),
            pltpu.VMEM((B, 16), jnp.int32),
        ],
    )(qn, kb)


def kernel(query_tokens, prompt_tokens, rag_tokens, embedding, W_score, b_score, kb):
    tok_dtype = query_tokens.dtype
    combined = jnp.concatenate([query_tokens[:, :256], rag_tokens], axis=1)
    comb32 = combined.astype(jnp.int32)

    emb = _sc_gather(embedding, comb32.reshape(-1)).reshape(B, SEQ, D)

    recon, scores, qn = _stage2(
        emb,
        comb32.reshape(B, 1, SEQ),
        W_score,
        b_score.reshape(1, 1),
    )

    idx = _stage3(qn, kb)

    return (
        recon.reshape(B, SEQ).astype(tok_dtype),
        scores.reshape(B, SEQ),
        idx[:, :TOP_K],
    )


# speedup vs baseline: 1.1151x; 1.1061x over previous
import functools

import jax
import jax.numpy as jnp
from jax import lax
from jax.experimental import pallas as pl
from jax.experimental.pallas import tpu as pltpu
from jax.experimental.pallas import tpu_sc as plsc

B = 32
SEQ = 512
D = 512
KB_N = 100000
TOP_N = 5
TOP_K = 5

_NEG = -3.0e38
_BIGI = 2 ** 30


def _sc_gather(table, idx):
    n = idx.shape[0]
    nw = 32
    per_w = n // nw
    ch = 128
    mesh = plsc.VectorSubcoreMesh(core_axis_name="c", subcore_axis_name="s")

    @functools.partial(
        pl.kernel,
        mesh=mesh,
        out_type=jax.ShapeDtypeStruct((n, D), jnp.float32),
        scratch_types=[
            pltpu.VMEM((ch,), jnp.int32),
            pltpu.VMEM((ch, D), jnp.float32),
            pltpu.SemaphoreType.DMA,
        ],
    )
    def k(table_hbm, idx_hbm, out_hbm, idx_v, rows_v, sem):
        wid = lax.axis_index("s") * 2 + lax.axis_index("c")
        base = wid * per_w
        for j in range(per_w // ch):
            off = base + j * ch
            pltpu.sync_copy(idx_hbm.at[pl.ds(off, ch)], idx_v)
            pltpu.async_copy(table_hbm.at[idx_v], rows_v, sem).wait()
            pltpu.sync_copy(rows_v, out_hbm.at[pl.ds(off, ch)])

    return k(table, idx)


def _lane_sum_512(x):
    c = ((x[:, 0:128] + x[:, 128:256]) + x[:, 256:384]) + x[:, 384:512]
    acc = c[:, 0:8]
    for k in range(1, 16):
        acc = acc + c[:, 8 * k:8 * (k + 1)]
    h = acc[:, 0:4] + acc[:, 4:8]
    h = h[:, 0:2] + h[:, 2:4]
    return h[:, 0:1] + h[:, 1:2]


def _lane_sum_512_T(x):
    r = x.shape[0]
    c = ((x[:, 0:128] + x[:, 128:256]) + x[:, 256:384]) + x[:, 384:512]
    ct = jnp.transpose(c)
    acc = ct[0:8, :]
    for k in range(1, 16):
        acc = acc + ct[8 * k:8 * (k + 1), :]
    h = acc[0:4, :] + acc[4:8, :]
    h = h[0:2, :] + h[2:4, :]
    tot = h[0:1, :] + h[1:2, :]
    back = jnp.transpose(jnp.broadcast_to(tot, (8, r)))
    return back[:, 0:1]


def _col_to_row(x):
    return jnp.transpose(jnp.broadcast_to(x, (x.shape[0], 8)))[0:1, :]


def _row_to_col(x):
    return jnp.transpose(jnp.broadcast_to(x, (8, x.shape[1])))[:, 0:1]


def _stage2_compute(emb_ref, tok_ref, w_ref, b_ref, recon_ref, score_ref, qn_ref):
    e = emb_ref[0]
    g = lax.dot_general(e, e, (((1,), (1,)), ((), ())),
                        preferred_element_type=jnp.float32)
    rowi = lax.broadcasted_iota(jnp.int32, (SEQ, SEQ), 0)
    coli = lax.broadcasted_iota(jnp.int32, (SEQ, SEQ), 1)
    eye = rowi == coli
    sq_col = _lane_sum_512_T(e * e)
    sq_row = _col_to_row(sq_col)
    d2 = sq_col + sq_row - 2.0 * g

    colf = coli.astype(jnp.float32)
    dsel = []
    work = d2
    for _ in range(TOP_N):
        m = jnp.min(work, axis=1, keepdims=True)
        dsel.append(jnp.sqrt(jnp.maximum(m, 0.0)))
        first = jnp.min(jnp.where(work <= m, colf, 3.0e38), axis=1, keepdims=True)
        work = jnp.where(colf == first, 3.0e38, work)
    mean_dist = (((dsel[0] + dsel[4]) + dsel[2]) + (dsel[1] + dsel[3])
                 ) * jnp.float32(0.2)

    lin = lax.dot_general(e, w_ref[...], (((1,), (0,)), ((), ())),
                          preferred_element_type=jnp.float32) + b_ref[0, 0]
    s_col = jax.nn.sigmoid(lin) * jnp.exp(-mean_dist)

    s_row = _col_to_row(s_col)
    tok_row = tok_ref[0]
    tok_col = _row_to_col(tok_row)

    beats = (s_row > s_col) | ((s_row == s_col) & (coli < rowi))
    rank_col = jnp.sum(beats.astype(jnp.int32), axis=1, keepdims=True)

    recon_row = jnp.sum(jnp.where(rank_col == coli, tok_col, 0),
                        axis=0, keepdims=True)

    acc8 = e[0:8, :]
    for k in range(1, 64):
        acc8 = acc8 + e[8 * k:8 * (k + 1), :]
    acc4 = acc8[0:4, :] + acc8[4:8, :]
    acc2 = acc4[0:2, :] + acc4[2:4, :]
    qm = (acc2[0:1, :] + acc2[1:2, :]) * jnp.float32(1.0 / 512.0)
    nrm = jnp.sqrt(_lane_sum_512(qm * qm))
    qn = qm / (nrm + 1e-8)

    recon_ref[0] = recon_row
    score_ref[0] = s_row
    qn_ref[pl.ds(pl.program_id(0), 1), :] = qn


_CHUNK = 5000
_NCHUNK = KB_N // _CHUNK


def _stage3_compute(c, qn, kb_ref, idx_out_ref, bv_ref, bi_ref):
    @pl.when(c == 0)
    def _init():
        bv_ref[...] = jnp.full((B, 16), _NEG, jnp.float32)
        bi_ref[...] = jnp.zeros((B, 16), jnp.int32)

    kbc = kb_ref[...]
    n2 = _lane_sum_512_T(kbc * kbc)
    kbn = kbc / (jnp.sqrt(n2) + 1e-8)
    s = lax.dot_general(qn, kbn, (((1,), (1,)), ((), ())),
                        preferred_element_type=jnp.float32)

    colf = lax.broadcasted_iota(jnp.int32, (B, _CHUNK), 1).astype(jnp.float32)
    lane16 = lax.broadcasted_iota(jnp.int32, (B, 16), 1)

    cand_v = bv_ref[...]
    cand_i = bi_ref[...]
    base = c * _CHUNK
    for t in range(TOP_K):
        m = jnp.max(s, axis=1, keepdims=True)
        first = jnp.min(jnp.where(s >= m, colf, 3.0e38), axis=1, keepdims=True)
        s = jnp.where(colf == first, _NEG, s)
        cand_v = jnp.where(lane16 == 8 + t, m, cand_v)
        cand_i = jnp.where(lane16 == 8 + t,
                           base + first.astype(jnp.int32), cand_i)

    res_v = jnp.full((B, 16), _NEG, jnp.float32)
    res_i = jnp.zeros((B, 16), jnp.int32)
    work = cand_v
    for t in range(TOP_K):
        m = jnp.max(work, axis=1, keepdims=True)
        sel = jnp.min(jnp.where(work >= m, cand_i, _BIGI), axis=1, keepdims=True)
        res_v = jnp.where(lane16 == t, m, res_v)
        res_i = jnp.where(lane16 == t, sel, res_i)
        work = jnp.where((work >= m) & (cand_i == sel), _NEG, work)
    bv_ref[...] = res_v
    bi_ref[...] = res_i
    idx_out_ref[...] = res_i


def _stage2_body(emb_ref, tok_ref, w_ref, b_ref, recon_ref, score_ref, qn_ref):
    _stage2_compute(emb_ref, tok_ref, w_ref, b_ref, recon_ref, score_ref, qn_ref)


def _stage2(emb, tok3, w, b11):
    return pl.pallas_call(
        _stage2_body,
        grid=(B,),
        in_specs=[
            pl.BlockSpec((1, SEQ, D), lambda b: (b, 0, 0)),
            pl.BlockSpec((1, 1, SEQ), lambda b: (b, 0, 0)),
            pl.BlockSpec((D, 1), lambda b: (0, 0)),
            pl.BlockSpec((1, 1), lambda b: (0, 0)),
        ],
        out_specs=[
            pl.BlockSpec((1, 1, SEQ), lambda b: (b, 0, 0)),
            pl.BlockSpec((1, 1, SEQ), lambda b: (b, 0, 0)),
            pl.BlockSpec((B, D), lambda b: (0, 0)),
        ],
        out_shape=[
            jax.ShapeDtypeStruct((B, 1, SEQ), jnp.int32),
            jax.ShapeDtypeStruct((B, 1, SEQ), jnp.float32),
            jax.ShapeDtypeStruct((B, D), jnp.float32),
        ],
    )(emb, tok3, w, b11)


def _stage3_body(qn_ref, kb_ref, idx_out_ref, bv_ref, bi_ref):
    _stage3_compute(pl.program_id(0), qn_ref[...], kb_ref,
                    idx_out_ref, bv_ref, bi_ref)


def _stage3(qn, kb):
    return pl.pallas_call(
        _stage3_body,
        grid=(_NCHUNK,),
        in_specs=[
            pl.BlockSpec((B, D), lambda c: (0, 0)),
            pl.BlockSpec((_CHUNK, D), lambda c: (c, 0)),
        ],
        out_specs=pl.BlockSpec((B, 16), lambda c: (0, 0)),
        out_shape=jax.ShapeDtypeStruct((B, 16), jnp.int32),
        scratch_shapes=[
            pltpu.VMEM((B, 16), jnp.float32),
            pltpu.VMEM((B, 16), jnp.int32),
        ],
    )(qn, kb)


def kernel(query_tokens, prompt_tokens, rag_tokens, embedding, W_score, b_score, kb):
    tok_dtype = query_tokens.dtype
    combined = jnp.concatenate([query_tokens[:, :256], rag_tokens], axis=1)
    comb32 = combined.astype(jnp.int32)

    emb = _sc_gather(embedding, comb32.reshape(-1)).reshape(B, SEQ, D)

    recon, scores, qn = _stage2(
        emb,
        comb32.reshape(B, 1, SEQ),
        W_score,
        b_score.reshape(1, 1),
    )

    idx = _stage3(qn, kb)

    return (
        recon.reshape(B, SEQ).astype(tok_dtype),
        scores.reshape(B, SEQ),
        idx[:, :TOP_K],
    )


# speedup vs baseline: 1.1152x; 1.0000x over previous
import functools

import jax
import jax.numpy as jnp
from jax import lax
from jax.experimental import pallas as pl
from jax.experimental.pallas import tpu as pltpu
from jax.experimental.pallas import tpu_sc as plsc

B = 32
SEQ = 512
D = 512
KB_N = 100000
TOP_N = 5
TOP_K = 5

_NEG = -3.0e38
_BIGI = 2 ** 30


def _sc_gather(table, idx):
    n = idx.shape[0]
    nw = 32
    per_w = n // nw
    ch = 128
    mesh = plsc.VectorSubcoreMesh(core_axis_name="c", subcore_axis_name="s")

    @functools.partial(
        pl.kernel,
        mesh=mesh,
        out_type=jax.ShapeDtypeStruct((n, D), jnp.float32),
        scratch_types=[
            pltpu.VMEM((ch,), jnp.int32),
            pltpu.VMEM((ch, D), jnp.float32),
            pltpu.SemaphoreType.DMA,
        ],
    )
    def k(table_hbm, idx_hbm, out_hbm, idx_v, rows_v, sem):
        wid = lax.axis_index("s") * 2 + lax.axis_index("c")
        base = wid * per_w
        for j in range(per_w // ch):
            off = base + j * ch
            pltpu.sync_copy(idx_hbm.at[pl.ds(off, ch)], idx_v)
            pltpu.async_copy(table_hbm.at[idx_v], rows_v, sem).wait()
            pltpu.sync_copy(rows_v, out_hbm.at[pl.ds(off, ch)])

    return k(table, idx)


def _lane_sum_512(x):
    c = ((x[:, 0:128] + x[:, 128:256]) + x[:, 256:384]) + x[:, 384:512]
    acc = c[:, 0:8]
    for k in range(1, 16):
        acc = acc + c[:, 8 * k:8 * (k + 1)]
    h = acc[:, 0:4] + acc[:, 4:8]
    h = h[:, 0:2] + h[:, 2:4]
    return h[:, 0:1] + h[:, 1:2]


def _lane_sum_512_T(x):
    r = x.shape[0]
    c = ((x[:, 0:128] + x[:, 128:256]) + x[:, 256:384]) + x[:, 384:512]
    ct = jnp.transpose(c)
    acc = ct[0:8, :]
    for k in range(1, 16):
        acc = acc + ct[8 * k:8 * (k + 1), :]
    h = acc[0:4, :] + acc[4:8, :]
    h = h[0:2, :] + h[2:4, :]
    tot = h[0:1, :] + h[1:2, :]
    back = jnp.transpose(jnp.broadcast_to(tot, (8, r)))
    return back[:, 0:1]


def _col_to_row(x):
    return jnp.transpose(jnp.broadcast_to(x, (x.shape[0], 8)))[0:1, :]


def _row_to_col(x):
    return jnp.transpose(jnp.broadcast_to(x, (8, x.shape[1])))[:, 0:1]


def _stage2_compute(emb_ref, tok_ref, w_ref, b_ref, recon_ref, score_ref, qn_ref):
    e = emb_ref[0]
    g = lax.dot_general(e, e, (((1,), (1,)), ((), ())),
                        preferred_element_type=jnp.float32)
    rowi = lax.broadcasted_iota(jnp.int32, (SEQ, SEQ), 0)
    coli = lax.broadcasted_iota(jnp.int32, (SEQ, SEQ), 1)
    sq_col = _lane_sum_512_T(e * e)
    sq_row = _col_to_row(sq_col)
    d2 = sq_col + sq_row - 2.0 * g

    colf = coli.astype(jnp.float32)
    dsel = []
    work = d2
    for _ in range(TOP_N):
        m = jnp.min(work, axis=1, keepdims=True)
        dsel.append(jnp.sqrt(jnp.maximum(m, 0.0)))
        first = jnp.min(jnp.where(work <= m, colf, 3.0e38), axis=1, keepdims=True)
        work = jnp.where(colf == first, 3.0e38, work)
    mean_dist = (((dsel[0] + dsel[4]) + dsel[2]) + (dsel[1] + dsel[3])
                 ) * jnp.float32(0.2)

    lin = lax.dot_general(e, w_ref[...], (((1,), (0,)), ((), ())),
                          preferred_element_type=jnp.float32) + b_ref[0, 0]
    s_col = jax.nn.sigmoid(lin) * jnp.exp(-mean_dist)

    s_row = _col_to_row(s_col)
    tok_row = tok_ref[0]
    tok_col = _row_to_col(tok_row)

    beats = (s_row > s_col) | ((s_row == s_col) & (coli < rowi))
    rank_col = jnp.sum(beats.astype(jnp.int32), axis=1, keepdims=True)

    recon_row = jnp.sum(jnp.where(rank_col == coli, tok_col, 0),
                        axis=0, keepdims=True)

    acc8 = e[0:8, :]
    for k in range(1, 64):
        acc8 = acc8 + e[8 * k:8 * (k + 1), :]
    acc4 = acc8[0:4, :] + acc8[4:8, :]
    acc2 = acc4[0:2, :] + acc4[2:4, :]
    qm = (acc2[0:1, :] + acc2[1:2, :]) * jnp.float32(1.0 / 512.0)
    nrm = jnp.sqrt(_lane_sum_512(qm * qm))
    qn = qm / (nrm + 1e-8)

    recon_ref[0] = recon_row
    score_ref[0] = s_row
    qn_ref[pl.ds(pl.program_id(0), 1), :] = qn


_CHUNK = 5000
_NCHUNK = KB_N // _CHUNK


def _stage3_compute(c, qn, kb_ref, idx_out_ref, bv_ref, bi_ref):
    @pl.when(c == 0)
    def _init():
        bv_ref[...] = jnp.full((B, 16), _NEG, jnp.float32)
        bi_ref[...] = jnp.zeros((B, 16), jnp.int32)

    kbc = kb_ref[...]
    n2 = _lane_sum_512_T(kbc * kbc)
    kbn = kbc / (jnp.sqrt(n2) + 1e-8)
    s = lax.dot_general(qn, kbn, (((1,), (1,)), ((), ())),
                        preferred_element_type=jnp.float32)

    colf = lax.broadcasted_iota(jnp.int32, (B, _CHUNK), 1).astype(jnp.float32)
    lane16 = lax.broadcasted_iota(jnp.int32, (B, 16), 1)

    cand_v = bv_ref[...]
    cand_i = bi_ref[...]
    base = c * _CHUNK
    for t in range(TOP_K):
        m = jnp.max(s, axis=1, keepdims=True)
        first = jnp.min(jnp.where(s >= m, colf, 3.0e38), axis=1, keepdims=True)
        s = jnp.where(colf == first, _NEG, s)
        cand_v = jnp.where(lane16 == 8 + t, m, cand_v)
        cand_i = jnp.where(lane16 == 8 + t,
                           base + first.astype(jnp.int32), cand_i)

    res_v = jnp.full((B, 16), _NEG, jnp.float32)
    res_i = jnp.zeros((B, 16), jnp.int32)
    work = cand_v
    for t in range(TOP_K):
        m = jnp.max(work, axis=1, keepdims=True)
        sel = jnp.min(jnp.where(work >= m, cand_i, _BIGI), axis=1, keepdims=True)
        res_v = jnp.where(lane16 == t, m, res_v)
        res_i = jnp.where(lane16 == t, sel, res_i)
        work = jnp.where((work >= m) & (cand_i == sel), _NEG, work)
    bv_ref[...] = res_v
    bi_ref[...] = res_i
    idx_out_ref[...] = res_i


def _stage2_body(emb_ref, tok_ref, w_ref, b_ref, recon_ref, score_ref, qn_ref):
    _stage2_compute(emb_ref, tok_ref, w_ref, b_ref, recon_ref, score_ref, qn_ref)


def _stage2(emb, tok3, w, b11):
    return pl.pallas_call(
        _stage2_body,
        grid=(B,),
        in_specs=[
            pl.BlockSpec((1, SEQ, D), lambda b: (b, 0, 0)),
            pl.BlockSpec((1, 1, SEQ), lambda b: (b, 0, 0)),
            pl.BlockSpec((D, 1), lambda b: (0, 0)),
            pl.BlockSpec((1, 1), lambda b: (0, 0)),
        ],
        out_specs=[
            pl.BlockSpec((1, 1, SEQ), lambda b: (b, 0, 0)),
            pl.BlockSpec((1, 1, SEQ), lambda b: (b, 0, 0)),
            pl.BlockSpec((B, D), lambda b: (0, 0)),
        ],
        out_shape=[
            jax.ShapeDtypeStruct((B, 1, SEQ), jnp.int32),
            jax.ShapeDtypeStruct((B, 1, SEQ), jnp.float32),
            jax.ShapeDtypeStruct((B, D), jnp.float32),
        ],
    )(emb, tok3, w, b11)


def _stage3_body(qn_ref, kb_ref, idx_out_ref, bv_ref, bi_ref):
    _stage3_compute(pl.program_id(0), qn_ref[...], kb_ref,
                    idx_out_ref, bv_ref, bi_ref)


def _stage3(qn, kb):
    return pl.pallas_call(
        _stage3_body,
        grid=(_NCHUNK,),
        in_specs=[
            pl.BlockSpec((B, D), lambda c: (0, 0)),
            pl.BlockSpec((_CHUNK, D), lambda c: (c, 0)),
        ],
        out_specs=pl.BlockSpec((B, 16), lambda c: (0, 0)),
        out_shape=jax.ShapeDtypeStruct((B, 16), jnp.int32),
        scratch_shapes=[
            pltpu.VMEM((B, 16), jnp.float32),
            pltpu.VMEM((B, 16), jnp.int32),
        ],
    )(qn, kb)


def kernel(query_tokens, prompt_tokens, rag_tokens, embedding, W_score, b_score, kb):
    tok_dtype = query_tokens.dtype
    combined = jnp.concatenate([query_tokens[:, :256], rag_tokens], axis=1)
    comb32 = combined.astype(jnp.int32)

    emb = _sc_gather(embedding, comb32.reshape(-1)).reshape(B, SEQ, D)

    recon, scores, qn = _stage2(
        emb,
        comb32.reshape(B, 1, SEQ),
        W_score,
        b_score.reshape(1, 1),
    )

    idx = _stage3(qn, kb)

    return (
        recon.reshape(B, SEQ).astype(tok_dtype),
        scores.reshape(B, SEQ),
        idx[:, :TOP_K],
    )
